# baseline (speedup 1.0000x reference)
import dataclasses

import jax
import jax.numpy as jnp
from jax import lax
from jax.experimental import pallas as pl
from jax.experimental.pallas import tpu as pltpu
from jax.experimental.pallas import tpu_sc as plsc

B, S = 128, 200
N = B * S
D, DT = 256, 768
C, VC, U = 1000, 100000, 4096
NB, NT, TSD = 3, 7, 8
EPS = 1e-6

NSPLIT = 1
H = N // NSPLIT
TOK_BLK = 1024
H_BLKS = H // TOK_BLK
KF = 32

NC, NS = 2, 16
NW = NC * NS
PER_W = H // NW
CH = 80
NCH = PER_W // CH


def _pack_rows(p):
    pb = p.astype(jnp.bfloat16).astype(jnp.float32)
    bits = lax.bitcast_convert_type(pb, jnp.uint32)
    lo = bits[:, :D // 2] >> 16
    hi = bits[:, D // 2:] & jnp.uint32(0xFFFF0000)
    return lax.bitcast_convert_type(lo | hi, jnp.int32)


def _unpack_rows(w):
    u = lax.bitcast_convert_type(w, jnp.uint32)
    lo = lax.bitcast_convert_type(u << 16, jnp.float32)
    hi = lax.bitcast_convert_type(u & jnp.uint32(0xFFFF0000), jnp.float32)
    return jnp.concatenate([lo, hi], axis=1)


def _proj_body(colt_ref, textt_ref, wcol_ref, wtext_ref, bcol_ref, btext_ref,
               colp_ref, textp_ref):
    @pl.when(pl.program_id(0) == 0)
    def _():
        colp_ref[...] = _pack_rows(
            jnp.dot(colt_ref[...].astype(jnp.bfloat16),
                    wcol_ref[...].astype(jnp.bfloat16),
                    preferred_element_type=jnp.float32) + bcol_ref[...])

    textp_ref[...] = _pack_rows(
        jnp.dot(textt_ref[...].astype(jnp.bfloat16),
                wtext_ref[...].astype(jnp.bfloat16),
                preferred_element_type=jnp.float32) + btext_ref[...])


def _project_tables(col_emb_table, text_batch_emb, W_col, W_text, b_col, b_text):
    ub = U // 4
    return pl.pallas_call(
        _proj_body,
        grid=(4,),
        in_specs=[pl.BlockSpec((C, DT), lambda i: (0, 0)),
                  pl.BlockSpec((ub, DT), lambda i: (i, 0)),
                  pl.BlockSpec((DT, D), lambda i: (0, 0)),
                  pl.BlockSpec((DT, D), lambda i: (0, 0)),
                  pl.BlockSpec((1, D), lambda i: (0, 0)),
                  pl.BlockSpec((1, D), lambda i: (0, 0))],
        out_specs=(pl.BlockSpec((C, D // 2), lambda i: (0, 0)),
                   pl.BlockSpec((ub, D // 2), lambda i: (i, 0))),
        out_shape=(jax.ShapeDtypeStruct((C, D // 2), jnp.int32),
                   jax.ShapeDtypeStruct((U, D // 2), jnp.int32)),
    )(col_emb_table, text_batch_emb, W_col, W_text,
      b_col.reshape(1, D), b_text.reshape(1, D))


def _pipelined_gathers(tabs, idx_refs, outs, bufs, gsem, wsem, base):
    nt = len(tabs)

    def g(t, ci, b):
        return pltpu.make_async_copy(
            tabs[t].at[idx_refs[t].at[pl.ds(ci * CH, CH)]], bufs[t][b],
            gsem.at[t, b])

    def w(t, ci, b):
        return pltpu.make_async_copy(
            bufs[t][b], outs[t].at[pl.ds(base + ci * CH, CH)], wsem.at[t, b])

    for t in range(nt):
        g(t, 0, 0).start()
    for t in range(nt):
        g(t, 0, 0).wait()
        g(t, 1, 1).start()
        w(t, 0, 0).start()

    @pl.loop(0, (NCH - 2) // 2)
    def _(k):
        for b, delta in ((1, 1), (0, 2)):
            ci = 2 * k + delta
            for t in range(nt):
                g(t, ci, b).wait()
                w(t, ci - 1, 1 - b).wait()
                g(t, ci + 1, 1 - b).start()
                w(t, ci, b).start()

    for t in range(nt):
        g(t, NCH - 1, 1).wait()
        w(t, NCH - 2, 0).wait()
        w(t, NCH - 1, 1).start()
        w(t, NCH - 1, 1).wait()


_MESH = plsc.VectorSubcoreMesh(core_axis_name="c", subcore_axis_name="s")

CHS = 64
NCHS = 14
N_PAD = N + TOK_BLK


def _sc_cat_body(cat_tab, cat_ids, flags, cat_out, idx_v, flg_v, cidx_v,
                 cpos_v, buf0, buf1, gsem, wsem):
    wid = lax.axis_index("s") * NC + lax.axis_index("c")
    base = wid * PER_W
    pltpu.sync_copy(cat_ids.at[pl.ds(base, PER_W)], idx_v)
    pltpu.sync_copy(flags.at[pl.ds(base, PER_W)], flg_v)

    zero16 = jnp.zeros((16,), jnp.int32)
    trash16 = jnp.full((16,), N + wid, jnp.int32)
    for r in range(NCHS):
        for l in range(CHS // 16):
            cidx_v[r, pl.ds(l * 16, 16)] = zero16
            cpos_v[r, pl.ds(l * 16, 16)] = trash16

    def cbody(j, carry):
        f = flg_v[pl.ds(j * 16, 16)]
        ids = idx_v[pl.ds(j * 16, 16)]
        pos = base + j * 16 + lax.iota(jnp.int32, 16)
        lanes = carry + jnp.cumsum(f) - f
        m = f > 0
        plsc.store_scatter(cidx_v, [lanes >> 6, lanes & 63], ids, mask=m)
        plsc.store_scatter(cpos_v, [lanes >> 6, lanes & 63], pos, mask=m)
        return carry + jnp.sum(f)

    k = lax.fori_loop(0, PER_W // 16, cbody, jnp.int32(0))
    npairs = (k + 2 * CHS - 1) // (2 * CHS)

    def pbody(j, carry):
        for half, buf in ((0, buf0), (1, buf1)):
            ci = 2 * j + half
            pltpu.make_async_copy(cat_tab.at[cidx_v.at[ci]], buf,
                                  gsem.at[0, half]).start()
        for half, buf in ((0, buf0), (1, buf1)):
            ci = 2 * j + half
            pltpu.make_async_copy(cat_tab.at[cidx_v.at[ci]], buf,
                                  gsem.at[0, half]).wait()
            pltpu.make_async_copy(buf, cat_out.at[cpos_v.at[ci]],
                                  wsem.at[0, half]).start()
        for half, buf in ((0, buf0), (1, buf1)):
            pltpu.make_async_copy(buf, cat_out.at[cpos_v.at[2 * j + half]],
                                  wsem.at[0, half]).wait()
        return carry

    lax.fori_loop(0, npairs, pbody, jnp.int32(0))


_SC_CP = pltpu.CompilerParams()
if "needs_layout_passes" in pltpu.CompilerParams.__dataclass_fields__:
    _SC_CP = dataclasses.replace(_SC_CP, needs_layout_passes=False)


def _sc_cat_gather(cat_tab, cat_ids, flags):
    k = pl.kernel(
        _sc_cat_body,
        mesh=_MESH,
        compiler_params=_SC_CP,
        out_type=jax.ShapeDtypeStruct((N_PAD, DT), jnp.float32),
        scratch_types=[pltpu.VMEM((PER_W,), jnp.int32),
                       pltpu.VMEM((PER_W,), jnp.int32),
                       pltpu.VMEM((NCHS, CHS), jnp.int32),
                       pltpu.VMEM((NCHS, CHS), jnp.int32),
                       pltpu.VMEM((CHS, DT), jnp.float32),
                       pltpu.VMEM((CHS, DT), jnp.float32),
                       pltpu.SemaphoreType.DMA((1, 2)),
                       pltpu.SemaphoreType.DMA((1, 2))],
    )
    return k(cat_tab, cat_ids, flags)


def _sc_proj_body(colp_tab, textp_tab, col_ids, text_ids, col_out, text_out,
                  cidx_v, tidx_v, cbuf0, cbuf1, tbuf0, tbuf1, gsem, wsem):
    wid = lax.axis_index("s") * NC + lax.axis_index("c")
    base = wid * PER_W
    pltpu.sync_copy(col_ids.at[pl.ds(base, PER_W)], cidx_v)
    pltpu.sync_copy(text_ids.at[pl.ds(base, PER_W)], tidx_v)
    _pipelined_gathers((colp_tab, textp_tab), (cidx_v, tidx_v),
                       (col_out, text_out),
                       ((cbuf0, cbuf1), (tbuf0, tbuf1)), gsem, wsem, base)


def _sc_proj_gather(colp_tab, textp_tab, col_ids, text_ids):
    k = pl.kernel(
        _sc_proj_body,
        mesh=_MESH,
        out_type=[jax.ShapeDtypeStruct((H, D // 2), jnp.int32),
                  jax.ShapeDtypeStruct((H, D // 2), jnp.int32)],
        scratch_types=[pltpu.VMEM((PER_W,), jnp.int32),
                       pltpu.VMEM((PER_W,), jnp.int32),
                       pltpu.VMEM((CH, D // 2), jnp.int32),
                       pltpu.VMEM((CH, D // 2), jnp.int32),
                       pltpu.VMEM((CH, D // 2), jnp.int32),
                       pltpu.VMEM((CH, D // 2), jnp.int32),
                       pltpu.SemaphoreType.DMA((2, 2)),
                       pltpu.SemaphoreType.DMA((2, 2))],
    )
    return k(colp_tab, textp_tab, col_ids, text_ids)


def _final_body(cat_ref, colv_ref, textv_ref, pk_ref, wcat_ref, r_ref,
                scale_ref, out_ref):
    f32 = jnp.float32
    catd = jnp.dot(cat_ref[...].astype(jnp.bfloat16), wcat_ref[...],
                   preferred_element_type=f32)
    sel = jnp.dot(pk_ref[...], r_ref[...], preferred_element_type=f32)
    am4 = pk_ref[:, 17:18].astype(f32)
    am5 = pk_ref[:, 18:19].astype(f32)
    ipad = pk_ref[:, 19:20].astype(f32)
    x = (_unpack_rows(colv_ref[...]) + sel
         + jnp.where(am4 > 0.0, catd, 0.0)
         + am5 * _unpack_rows(textv_ref[...]))
    xc = x - jnp.mean(x, axis=1, keepdims=True)
    h = xc * lax.rsqrt(jnp.mean(xc * xc, axis=1, keepdims=True) + EPS)
    out_ref[...] = h * scale_ref[...] * (1.0 - ipad)


def _final(half, cat_rows, col_rows, text_rows, pk, wcat_bf, r_mat, rms_scale):
    tok = lambda w: pl.BlockSpec((TOK_BLK, w), lambda i: (i, 0))
    pk_spec = pl.BlockSpec((TOK_BLK, KF), lambda i: (i + half * H_BLKS, 0))
    rep = lambda a, b: pl.BlockSpec((a, b), lambda i: (0, 0))
    return pl.pallas_call(
        _final_body,
        grid=(H_BLKS,),
        in_specs=[tok(DT), tok(D // 2), tok(D // 2), pk_spec,
                  rep(DT, D), rep(KF, D), rep(1, D)],
        out_specs=pl.BlockSpec((TOK_BLK, D), lambda i: (i, 0)),
        out_shape=jax.ShapeDtypeStruct((H, D), jnp.float32),
    )(cat_rows, col_rows, text_rows, pk, wcat_bf, r_mat,
      rms_scale.reshape(1, D))


def kernel(semantic_types, column_ids, is_null, is_target, is_padding,
           numeric_values, timestamp_values, bool_values,
           categorical_embed_ids, text_embed_ids,
           col_emb_table, cat_emb_table, text_batch_emb,
           W_col, b_col, W_num, b_num, W_ts, b_ts, W_cat, b_cat,
           W_text, b_text, bool_emb_table, identifier_emb, null_emb,
           mask_emb, rms_scale):
    f32 = jnp.float32
    cat_ids = categorical_embed_ids.astype(jnp.int32).reshape(NSPLIT, H)
    col_ids = column_ids.astype(jnp.int32).reshape(NSPLIT, H)
    text_ids = text_embed_ids.astype(jnp.int32).reshape(NSPLIT, H)
    cat_need = ((semantic_types == 4) & (is_null == 0)
                & (is_target == 0)).astype(jnp.int32).reshape(NSPLIT, H)

    colp, textp = _project_tables(col_emb_table, text_batch_emb,
                                  W_col, W_text, b_col, b_text)

    st = semantic_types.astype(f32)
    inul = is_null.astype(f32)
    itgt = is_target.astype(f32)
    a = (1.0 - itgt) * (1.0 - inul)
    bmix = (1.0 - itgt) * inul
    m = [a * (st == k).astype(f32) for k in range(6)]
    boolc = jnp.clip(bool_values.astype(f32), 0.0, NB - 1.0)
    cols = [m[0],
            m[1] * numeric_values,
            m[1]]
    cols += [m[2] * timestamp_values[..., j] for j in range(TSD)]
    cols += [m[2],
             m[3] * (boolc == 0.0).astype(f32),
             m[3] * (boolc == 1.0).astype(f32),
             m[3] * (boolc == 2.0).astype(f32),
             bmix,
             itgt,
             m[4],
             m[5],
             is_padding.astype(f32)]
    pk = jnp.stack(cols + [jnp.zeros((B, S), f32)] * (KF - len(cols)),
                   axis=-1).astype(jnp.bfloat16).reshape(N, KF)

    r_mat = jnp.concatenate(
        [identifier_emb.reshape(1, D), W_num, b_num.reshape(1, D), W_ts,
         b_ts.reshape(1, D), bool_emb_table, null_emb.reshape(1, D),
         mask_emb.reshape(1, D), b_cat.reshape(1, D),
         jnp.zeros((KF - 18, D), f32)], axis=0).astype(jnp.bfloat16)
    wcat_bf = W_cat.astype(jnp.bfloat16)

    outs = []
    prev = None
    for h in range(NSPLIT):
        cids = cat_ids[h]
        if prev is not None:
            cids, _ = lax.optimization_barrier((cids, prev))
        cat_rows = _sc_cat_gather(cat_emb_table, cids, cat_need[h])
        colp_h, textp_h, cat_rows = lax.optimization_barrier(
            (colp, textp, cat_rows))
        col_rows, text_rows = _sc_proj_gather(colp_h, textp_h,
                                              col_ids[h], text_ids[h])
        prev = col_rows
        outs.append(_final(h, cat_rows, col_rows, text_rows, pk,
                           wcat_bf, r_mat, rms_scale))

    return jnp.concatenate(outs, axis=0).reshape(B, S, D)


# speedup vs baseline: 1.6017x; 1.6017x over previous
import dataclasses

import jax
import jax.numpy as jnp
from jax import lax
from jax.experimental import pallas as pl
from jax.experimental.pallas import tpu as pltpu
from jax.experimental.pallas import tpu_sc as plsc

B, S = 128, 200
N = B * S
D, DT = 256, 768
C, VC, U = 1000, 100000, 4096
NB, NT, TSD = 3, 7, 8
EPS = 1e-6

NSPLIT = 1
H = N // NSPLIT
TOK_BLK = 1024
H_BLKS = H // TOK_BLK
KF = 32

NC, NS = 2, 16
NW = NC * NS
PER_W = H // NW
CH = 80
NCH = PER_W // CH


def _pack_rows(p):
    pb = p.astype(jnp.bfloat16).astype(jnp.float32)
    bits = lax.bitcast_convert_type(pb, jnp.uint32)
    lo = bits[:, :D // 2] >> 16
    hi = bits[:, D // 2:] & jnp.uint32(0xFFFF0000)
    return lax.bitcast_convert_type(lo | hi, jnp.int32)


def _unpack_rows(w):
    u = lax.bitcast_convert_type(w, jnp.uint32)
    lo = lax.bitcast_convert_type(u << 16, jnp.float32)
    hi = lax.bitcast_convert_type(u & jnp.uint32(0xFFFF0000), jnp.float32)
    return jnp.concatenate([lo, hi], axis=1)


def _proj_body(colt_ref, textt_ref, wcol_ref, wtext_ref, bcol_ref, btext_ref,
               colp_ref, textp_ref):
    @pl.when(pl.program_id(0) == 0)
    def _():
        colp_ref[...] = _pack_rows(
            jnp.dot(colt_ref[...].astype(jnp.bfloat16),
                    wcol_ref[...].astype(jnp.bfloat16),
                    preferred_element_type=jnp.float32) + bcol_ref[...])

    textp_ref[...] = _pack_rows(
        jnp.dot(textt_ref[...].astype(jnp.bfloat16),
                wtext_ref[...].astype(jnp.bfloat16),
                preferred_element_type=jnp.float32) + btext_ref[...])


def _project_tables(col_emb_table, text_batch_emb, W_col, W_text, b_col, b_text):
    ub = U // 4
    return pl.pallas_call(
        _proj_body,
        grid=(4,),
        in_specs=[pl.BlockSpec((C, DT), lambda i: (0, 0)),
                  pl.BlockSpec((ub, DT), lambda i: (i, 0)),
                  pl.BlockSpec((DT, D), lambda i: (0, 0)),
                  pl.BlockSpec((DT, D), lambda i: (0, 0)),
                  pl.BlockSpec((1, D), lambda i: (0, 0)),
                  pl.BlockSpec((1, D), lambda i: (0, 0))],
        out_specs=(pl.BlockSpec((C, D // 2), lambda i: (0, 0)),
                   pl.BlockSpec((ub, D // 2), lambda i: (i, 0))),
        out_shape=(jax.ShapeDtypeStruct((C, D // 2), jnp.int32),
                   jax.ShapeDtypeStruct((U, D // 2), jnp.int32)),
    )(col_emb_table, text_batch_emb, W_col, W_text,
      b_col.reshape(1, D), b_text.reshape(1, D))


def _pipelined_gathers(tabs, idx_refs, outs, bufs, gsem, wsem, base):
    nt = len(tabs)

    def g(t, ci, b):
        return pltpu.make_async_copy(
            tabs[t].at[idx_refs[t].at[pl.ds(ci * CH, CH)]], bufs[t][b],
            gsem.at[t, b])

    def w(t, ci, b):
        return pltpu.make_async_copy(
            bufs[t][b], outs[t].at[pl.ds(base + ci * CH, CH)], wsem.at[t, b])

    for t in range(nt):
        g(t, 0, 0).start()
    for t in range(nt):
        g(t, 0, 0).wait()
        g(t, 1, 1).start()
        w(t, 0, 0).start()

    @pl.loop(0, (NCH - 2) // 2)
    def _(k):
        for b, delta in ((1, 1), (0, 2)):
            ci = 2 * k + delta
            for t in range(nt):
                g(t, ci, b).wait()
                w(t, ci - 1, 1 - b).wait()
                g(t, ci + 1, 1 - b).start()
                w(t, ci, b).start()

    for t in range(nt):
        g(t, NCH - 1, 1).wait()
        w(t, NCH - 2, 0).wait()
        w(t, NCH - 1, 1).start()
        w(t, NCH - 1, 1).wait()


_MESH = plsc.VectorSubcoreMesh(core_axis_name="c", subcore_axis_name="s")

CHS = 64
NCHS = 14
N_PAD = N + TOK_BLK


def _sc_cat_body(cat_tab, cat_ids, cat_out, idx_v, buf0, buf1, gsem, wsem):
    wid = lax.axis_index("s") * NC + lax.axis_index("c")
    base = wid * PER_W
    pltpu.sync_copy(cat_ids.at[pl.ds(base, PER_W)], idx_v)
    _pipelined_gathers((cat_tab,), (idx_v,), (cat_out,), ((buf0, buf1),),
                       gsem, wsem, base)


def _sc_cat_gather(cat_tab, cat_ids):
    k = pl.kernel(
        _sc_cat_body,
        mesh=_MESH,
        out_type=jax.ShapeDtypeStruct((H, DT), jnp.float32),
        scratch_types=[pltpu.VMEM((PER_W,), jnp.int32),
                       pltpu.VMEM((CH, DT), jnp.float32),
                       pltpu.VMEM((CH, DT), jnp.float32),
                       pltpu.SemaphoreType.DMA((1, 2)),
                       pltpu.SemaphoreType.DMA((1, 2))],
    )
    return k(cat_tab, cat_ids)


def _sc_proj_body(colp_tab, textp_tab, col_ids, text_ids, col_out, text_out,
                  cidx_v, tidx_v, cbuf0, cbuf1, tbuf0, tbuf1, gsem, wsem):
    wid = lax.axis_index("s") * NC + lax.axis_index("c")
    base = wid * PER_W
    pltpu.sync_copy(col_ids.at[pl.ds(base, PER_W)], cidx_v)
    pltpu.sync_copy(text_ids.at[pl.ds(base, PER_W)], tidx_v)
    _pipelined_gathers((colp_tab, textp_tab), (cidx_v, tidx_v),
                       (col_out, text_out),
                       ((cbuf0, cbuf1), (tbuf0, tbuf1)), gsem, wsem, base)


def _sc_proj_gather(colp_tab, textp_tab, col_ids, text_ids):
    k = pl.kernel(
        _sc_proj_body,
        mesh=_MESH,
        out_type=[jax.ShapeDtypeStruct((H, D // 2), jnp.int32),
                  jax.ShapeDtypeStruct((H, D // 2), jnp.int32)],
        scratch_types=[pltpu.VMEM((PER_W,), jnp.int32),
                       pltpu.VMEM((PER_W,), jnp.int32),
                       pltpu.VMEM((CH, D // 2), jnp.int32),
                       pltpu.VMEM((CH, D // 2), jnp.int32),
                       pltpu.VMEM((CH, D // 2), jnp.int32),
                       pltpu.VMEM((CH, D // 2), jnp.int32),
                       pltpu.SemaphoreType.DMA((2, 2)),
                       pltpu.SemaphoreType.DMA((2, 2))],
    )
    return k(colp_tab, textp_tab, col_ids, text_ids)


def _final_body(cat_ref, colv_ref, textv_ref, pk_ref, wcat_ref, r_ref,
                scale_ref, out_ref):
    f32 = jnp.float32
    catd = jnp.dot(cat_ref[...].astype(jnp.bfloat16), wcat_ref[...],
                   preferred_element_type=f32)
    sel = jnp.dot(pk_ref[...], r_ref[...], preferred_element_type=f32)
    am4 = pk_ref[:, 17:18].astype(f32)
    am5 = pk_ref[:, 18:19].astype(f32)
    ipad = pk_ref[:, 19:20].astype(f32)
    x = (_unpack_rows(colv_ref[...]) + sel
         + jnp.where(am4 > 0.0, catd, 0.0)
         + am5 * _unpack_rows(textv_ref[...]))
    xc = x - jnp.mean(x, axis=1, keepdims=True)
    h = xc * lax.rsqrt(jnp.mean(xc * xc, axis=1, keepdims=True) + EPS)
    out_ref[...] = h * scale_ref[...] * (1.0 - ipad)


def _final(half, cat_rows, col_rows, text_rows, pk, wcat_bf, r_mat, rms_scale):
    tok = lambda w: pl.BlockSpec((TOK_BLK, w), lambda i: (i, 0))
    pk_spec = pl.BlockSpec((TOK_BLK, KF), lambda i: (i + half * H_BLKS, 0))
    rep = lambda a, b: pl.BlockSpec((a, b), lambda i: (0, 0))
    return pl.pallas_call(
        _final_body,
        grid=(H_BLKS,),
        in_specs=[tok(DT), tok(D // 2), tok(D // 2), pk_spec,
                  rep(DT, D), rep(KF, D), rep(1, D)],
        out_specs=pl.BlockSpec((TOK_BLK, D), lambda i: (i, 0)),
        out_shape=jax.ShapeDtypeStruct((H, D), jnp.float32),
    )(cat_rows, col_rows, text_rows, pk, wcat_bf, r_mat,
      rms_scale.reshape(1, D))


def kernel(semantic_types, column_ids, is_null, is_target, is_padding,
           numeric_values, timestamp_values, bool_values,
           categorical_embed_ids, text_embed_ids,
           col_emb_table, cat_emb_table, text_batch_emb,
           W_col, b_col, W_num, b_num, W_ts, b_ts, W_cat, b_cat,
           W_text, b_text, bool_emb_table, identifier_emb, null_emb,
           mask_emb, rms_scale):
    f32 = jnp.float32
    cat_ids = categorical_embed_ids.astype(jnp.int32).reshape(NSPLIT, H)
    col_ids = column_ids.astype(jnp.int32).reshape(NSPLIT, H)
    text_ids = text_embed_ids.astype(jnp.int32).reshape(NSPLIT, H)

    colp, textp = _project_tables(col_emb_table, text_batch_emb,
                                  W_col, W_text, b_col, b_text)

    st = semantic_types.astype(f32)
    inul = is_null.astype(f32)
    itgt = is_target.astype(f32)
    a = (1.0 - itgt) * (1.0 - inul)
    bmix = (1.0 - itgt) * inul
    m = [a * (st == k).astype(f32) for k in range(6)]
    boolc = jnp.clip(bool_values.astype(f32), 0.0, NB - 1.0)
    cols = [m[0],
            m[1] * numeric_values,
            m[1]]
    cols += [m[2] * timestamp_values[..., j] for j in range(TSD)]
    cols += [m[2],
             m[3] * (boolc == 0.0).astype(f32),
             m[3] * (boolc == 1.0).astype(f32),
             m[3] * (boolc == 2.0).astype(f32),
             bmix,
             itgt,
             m[4],
             m[5],
             is_padding.astype(f32)]
    pk = jnp.stack(cols + [jnp.zeros((B, S), f32)] * (KF - len(cols)),
                   axis=-1).astype(jnp.bfloat16).reshape(N, KF)

    r_mat = jnp.concatenate(
        [identifier_emb.reshape(1, D), W_num, b_num.reshape(1, D), W_ts,
         b_ts.reshape(1, D), bool_emb_table, null_emb.reshape(1, D),
         mask_emb.reshape(1, D), b_cat.reshape(1, D),
         jnp.zeros((KF - 18, D), f32)], axis=0).astype(jnp.bfloat16)
    wcat_bf = W_cat.astype(jnp.bfloat16)

    outs = []
    prev = None
    for h in range(NSPLIT):
        cids = cat_ids[h]
        if prev is not None:
            cids, _ = lax.optimization_barrier((cids, prev))
        cat_rows = _sc_cat_gather(cat_emb_table, cids)
        colp_h, textp_h, cat_rows = lax.optimization_barrier(
            (colp, textp, cat_rows))
        col_rows, text_rows = _sc_proj_gather(colp_h, textp_h,
                                              col_ids[h], text_ids[h])
        prev = col_rows
        outs.append(_final(h, cat_rows, col_rows, text_rows, pk,
                           wcat_bf, r_mat, rms_scale))

    return jnp.concatenate(outs, axis=0).reshape(B, S, D)


# speedup vs baseline: 1.6729x; 1.0444x over previous
import dataclasses

import jax
import jax.numpy as jnp
from jax import lax
from jax.experimental import pallas as pl
from jax.experimental.pallas import tpu as pltpu
from jax.experimental.pallas import tpu_sc as plsc

B, S = 128, 200
N = B * S
D, DT = 256, 768
C, VC, U = 1000, 100000, 4096
NB, NT, TSD = 3, 7, 8
EPS = 1e-6

NSPLIT = 1
H = N // NSPLIT
TOK_BLK = 2048
H_BLKS = H // TOK_BLK
KF = 32

NC, NS = 2, 16
NW = NC * NS
PER_W = H // NW
CH = 80
NCH = PER_W // CH


def _pack_rows(p):
    pb = p.astype(jnp.bfloat16).astype(jnp.float32)
    bits = lax.bitcast_convert_type(pb, jnp.uint32)
    lo = bits[:, :D // 2] >> 16
    hi = bits[:, D // 2:] & jnp.uint32(0xFFFF0000)
    return lax.bitcast_convert_type(lo | hi, jnp.int32)


def _unpack_rows(w):
    u = lax.bitcast_convert_type(w, jnp.uint32)
    lo = lax.bitcast_convert_type(u << 16, jnp.float32)
    hi = lax.bitcast_convert_type(u & jnp.uint32(0xFFFF0000), jnp.float32)
    return jnp.concatenate([lo, hi], axis=1)


def _proj_body(colt_ref, textt_ref, wcol_ref, wtext_ref, bcol_ref, btext_ref,
               colp_ref, textp_ref):
    @pl.when(pl.program_id(0) == 0)
    def _():
        colp_ref[...] = _pack_rows(
            jnp.dot(colt_ref[...].astype(jnp.bfloat16),
                    wcol_ref[...].astype(jnp.bfloat16),
                    preferred_element_type=jnp.float32) + bcol_ref[...])

    textp_ref[...] = _pack_rows(
        jnp.dot(textt_ref[...].astype(jnp.bfloat16),
                wtext_ref[...].astype(jnp.bfloat16),
                preferred_element_type=jnp.float32) + btext_ref[...])


def _project_tables(col_emb_table, text_batch_emb, W_col, W_text, b_col, b_text):
    ub = U // 4
    return pl.pallas_call(
        _proj_body,
        grid=(4,),
        in_specs=[pl.BlockSpec((C, DT), lambda i: (0, 0)),
                  pl.BlockSpec((ub, DT), lambda i: (i, 0)),
                  pl.BlockSpec((DT, D), lambda i: (0, 0)),
                  pl.BlockSpec((DT, D), lambda i: (0, 0)),
                  pl.BlockSpec((1, D), lambda i: (0, 0)),
                  pl.BlockSpec((1, D), lambda i: (0, 0))],
        out_specs=(pl.BlockSpec((C, D // 2), lambda i: (0, 0)),
                   pl.BlockSpec((ub, D // 2), lambda i: (i, 0))),
        out_shape=(jax.ShapeDtypeStruct((C, D // 2), jnp.int32),
                   jax.ShapeDtypeStruct((U, D // 2), jnp.int32)),
    )(col_emb_table, text_batch_emb, W_col, W_text,
      b_col.reshape(1, D), b_text.reshape(1, D))


def _pipelined_gathers(tabs, idx_refs, outs, bufs, gsem, wsem, base):
    nt = len(tabs)

    def g(t, ci, b):
        return pltpu.make_async_copy(
            tabs[t].at[idx_refs[t].at[pl.ds(ci * CH, CH)]], bufs[t][b],
            gsem.at[t, b])

    def w(t, ci, b):
        return pltpu.make_async_copy(
            bufs[t][b], outs[t].at[pl.ds(base + ci * CH, CH)], wsem.at[t, b])

    for t in range(nt):
        g(t, 0, 0).start()
    for t in range(nt):
        g(t, 0, 0).wait()
        g(t, 1, 1).start()
        w(t, 0, 0).start()

    @pl.loop(0, (NCH - 2) // 2)
    def _(k):
        for b, delta in ((1, 1), (0, 2)):
            ci = 2 * k + delta
            for t in range(nt):
                g(t, ci, b).wait()
                w(t, ci - 1, 1 - b).wait()
                g(t, ci + 1, 1 - b).start()
                w(t, ci, b).start()

    for t in range(nt):
        g(t, NCH - 1, 1).wait()
        w(t, NCH - 2, 0).wait()
        w(t, NCH - 1, 1).start()
        w(t, NCH - 1, 1).wait()


_MESH = plsc.VectorSubcoreMesh(core_axis_name="c", subcore_axis_name="s")

CHS = 64
NCHS = 14
N_PAD = N + TOK_BLK


def _sc_cat_body(cat_tab, cat_ids, cat_out, idx_v, buf0, buf1, gsem, wsem):
    wid = lax.axis_index("s") * NC + lax.axis_index("c")
    base = wid * PER_W
    pltpu.sync_copy(cat_ids.at[pl.ds(base, PER_W)], idx_v)
    _pipelined_gathers((cat_tab,), (idx_v,), (cat_out,), ((buf0, buf1),),
                       gsem, wsem, base)


def _sc_cat_gather(cat_tab, cat_ids):
    k = pl.kernel(
        _sc_cat_body,
        mesh=_MESH,
        out_type=jax.ShapeDtypeStruct((H, DT), jnp.float32),
        scratch_types=[pltpu.VMEM((PER_W,), jnp.int32),
                       pltpu.VMEM((CH, DT), jnp.float32),
                       pltpu.VMEM((CH, DT), jnp.float32),
                       pltpu.SemaphoreType.DMA((1, 2)),
                       pltpu.SemaphoreType.DMA((1, 2))],
    )
    return k(cat_tab, cat_ids)


def _sc_proj_body(colp_tab, textp_tab, col_ids, text_ids, col_out, text_out,
                  cidx_v, tidx_v, cbuf0, cbuf1, tbuf0, tbuf1, gsem, wsem):
    wid = lax.axis_index("s") * NC + lax.axis_index("c")
    base = wid * PER_W
    pltpu.sync_copy(col_ids.at[pl.ds(base, PER_W)], cidx_v)
    pltpu.sync_copy(text_ids.at[pl.ds(base, PER_W)], tidx_v)
    _pipelined_gathers((colp_tab, textp_tab), (cidx_v, tidx_v),
                       (col_out, text_out),
                       ((cbuf0, cbuf1), (tbuf0, tbuf1)), gsem, wsem, base)


def _sc_proj_gather(colp_tab, textp_tab, col_ids, text_ids):
    k = pl.kernel(
        _sc_proj_body,
        mesh=_MESH,
        out_type=[jax.ShapeDtypeStruct((H, D // 2), jnp.int32),
                  jax.ShapeDtypeStruct((H, D // 2), jnp.int32)],
        scratch_types=[pltpu.VMEM((PER_W,), jnp.int32),
                       pltpu.VMEM((PER_W,), jnp.int32),
                       pltpu.VMEM((CH, D // 2), jnp.int32),
                       pltpu.VMEM((CH, D // 2), jnp.int32),
                       pltpu.VMEM((CH, D // 2), jnp.int32),
                       pltpu.VMEM((CH, D // 2), jnp.int32),
                       pltpu.SemaphoreType.DMA((2, 2)),
                       pltpu.SemaphoreType.DMA((2, 2))],
    )
    return k(colp_tab, textp_tab, col_ids, text_ids)


def _final_body(cat_ref, colv_ref, textv_ref, pk_ref, wcat_ref, r_ref,
                scale_ref, out_ref):
    f32 = jnp.float32
    catd = jnp.dot(cat_ref[...].astype(jnp.bfloat16), wcat_ref[...],
                   preferred_element_type=f32)
    sel = jnp.dot(pk_ref[...], r_ref[...], preferred_element_type=f32)
    am4 = pk_ref[:, 17:18].astype(f32)
    am5 = pk_ref[:, 18:19].astype(f32)
    ipad = pk_ref[:, 19:20].astype(f32)
    x = (_unpack_rows(colv_ref[...]) + sel
         + jnp.where(am4 > 0.0, catd, 0.0)
         + am5 * _unpack_rows(textv_ref[...]))
    xc = x - jnp.mean(x, axis=1, keepdims=True)
    h = xc * lax.rsqrt(jnp.mean(xc * xc, axis=1, keepdims=True) + EPS)
    out_ref[...] = h * scale_ref[...] * (1.0 - ipad)


def _final(half, cat_rows, col_rows, text_rows, pk, wcat_bf, r_mat, rms_scale):
    tok = lambda w: pl.BlockSpec((TOK_BLK, w), lambda i: (i, 0))
    pk_spec = pl.BlockSpec((TOK_BLK, KF), lambda i: (i + half * H_BLKS, 0))
    rep = lambda a, b: pl.BlockSpec((a, b), lambda i: (0, 0))
    return pl.pallas_call(
        _final_body,
        grid=(H_BLKS,),
        in_specs=[tok(DT), tok(D // 2), tok(D // 2), pk_spec,
                  rep(DT, D), rep(KF, D), rep(1, D)],
        out_specs=pl.BlockSpec((TOK_BLK, D), lambda i: (i, 0)),
        out_shape=jax.ShapeDtypeStruct((H, D), jnp.float32),
    )(cat_rows, col_rows, text_rows, pk, wcat_bf, r_mat,
      rms_scale.reshape(1, D))


def kernel(semantic_types, column_ids, is_null, is_target, is_padding,
           numeric_values, timestamp_values, bool_values,
           categorical_embed_ids, text_embed_ids,
           col_emb_table, cat_emb_table, text_batch_emb,
           W_col, b_col, W_num, b_num, W_ts, b_ts, W_cat, b_cat,
           W_text, b_text, bool_emb_table, identifier_emb, null_emb,
           mask_emb, rms_scale):
    f32 = jnp.float32
    cat_ids = categorical_embed_ids.astype(jnp.int32).reshape(NSPLIT, H)
    col_ids = column_ids.astype(jnp.int32).reshape(NSPLIT, H)
    text_ids = text_embed_ids.astype(jnp.int32).reshape(NSPLIT, H)

    colp, textp = _project_tables(col_emb_table, text_batch_emb,
                                  W_col, W_text, b_col, b_text)

    st = semantic_types.astype(f32)
    inul = is_null.astype(f32)
    itgt = is_target.astype(f32)
    a = (1.0 - itgt) * (1.0 - inul)
    bmix = (1.0 - itgt) * inul
    m = [a * (st == k).astype(f32) for k in range(6)]
    boolc = jnp.clip(bool_values.astype(f32), 0.0, NB - 1.0)
    cols = [m[0],
            m[1] * numeric_values,
            m[1]]
    cols += [m[2] * timestamp_values[..., j] for j in range(TSD)]
    cols += [m[2],
             m[3] * (boolc == 0.0).astype(f32),
             m[3] * (boolc == 1.0).astype(f32),
             m[3] * (boolc == 2.0).astype(f32),
             bmix,
             itgt,
             m[4],
             m[5],
             is_padding.astype(f32)]
    pk = jnp.stack(cols + [jnp.zeros((B, S), f32)] * (KF - len(cols)),
                   axis=-1).astype(jnp.bfloat16).reshape(N, KF)

    r_mat = jnp.concatenate(
        [identifier_emb.reshape(1, D), W_num, b_num.reshape(1, D), W_ts,
         b_ts.reshape(1, D), bool_emb_table, null_emb.reshape(1, D),
         mask_emb.reshape(1, D), b_cat.reshape(1, D),
         jnp.zeros((KF - 18, D), f32)], axis=0).astype(jnp.bfloat16)
    wcat_bf = W_cat.astype(jnp.bfloat16)

    outs = []
    prev = None
    for h in range(NSPLIT):
        cids = cat_ids[h]
        if prev is not None:
            cids, _ = lax.optimization_barrier((cids, prev))
        cat_rows = _sc_cat_gather(cat_emb_table, cids)
        colp_h, textp_h, cat_rows = lax.optimization_barrier(
            (colp, textp, cat_rows))
        col_rows, text_rows = _sc_proj_gather(colp_h, textp_h,
                                              col_ids[h], text_ids[h])
        prev = col_rows
        outs.append(_final(h, cat_rows, col_rows, text_rows, pk,
                           wcat_bf, r_mat, rms_scale))

    return jnp.concatenate(outs, axis=0).reshape(B, S, D)
